# Initial kernel scaffold; baseline (speedup 1.0000x reference)
import jax
import jax.numpy as jnp
from jax.experimental import pallas as pl


def kernel(image, flow):
    raise NotImplementedError("write your pallas kernel here")


# speedup vs baseline: 1.7024x; 1.7024x over previous
import functools

import jax
import jax.numpy as jnp
from jax import lax
from jax.experimental import pallas as pl
from jax.experimental.pallas import tpu as pltpu
from jax.experimental.pallas import tpu_sc as plsc


B = 2
C = 2
H = 128
W = 128
D = 128
HWD = H * W * D
N = B * HWD
RZ = D // 2
RW = 8
NROW = B * H * W * RZ

NC = 2
NS = 16
NW = NC * NS

PW = N // NW
K = 1024
KC = K // 128
NCHUNK = PW // K
GROUPS = K // 16


def _floor_i32(x):
  t = x.astype(jnp.int32)
  tf = t.astype(jnp.float32)
  return t - jnp.where(tf > x, 1, 0).astype(jnp.int32)


def _warp_body(table_hbm, flow_hbm, out_hbm,
               dx_v, dy_v, dz_v,
               wx_v, wy_v, wz_v, cb_v,
               idx_v, rows_v,
               o0_v, o1_v, sem):
  cid = lax.axis_index("c")
  sid = lax.axis_index("s")
  wid = sid * NC + cid
  b = wid // (NW // B)
  obase = wid * PW - b * HWD
  bhw = b * H * W

  def chunk_body(j, _):
    o0 = obase + j * K
    fbase = b * 3 * HWD + o0
    pltpu.sync_copy(flow_hbm.at[pl.ds(fbase + 0 * HWD, K)], dy_v)
    pltpu.sync_copy(flow_hbm.at[pl.ds(fbase + 1 * HWD, K)], dx_v)
    pltpu.sync_copy(flow_hbm.at[pl.ds(fbase + 2 * HWD, K)], dz_v)

    def grp_idx(g, _):
      s = g * 16
      o = o0 + s + lax.iota(jnp.int32, 16)
      hh = (o >> 14) & 127
      ww = (o >> 7) & 127
      dd = o & 127
      x = ww.astype(jnp.float32) + dx_v[pl.ds(s, 16)]
      y = hh.astype(jnp.float32) + dy_v[pl.ds(s, 16)]
      z = dd.astype(jnp.float32) + dz_v[pl.ds(s, 16)]
      xf = _floor_i32(x)
      yf = _floor_i32(y)
      zf = _floor_i32(z)
      x0 = jnp.clip(xf, 0, W - 1)
      x1 = jnp.clip(xf + 1, 0, W - 1)
      y0 = jnp.clip(yf, 0, H - 1)
      y1 = jnp.clip(yf + 1, 0, H - 1)
      z0 = jnp.clip(zf, 0, D - 1)
      wx_v[pl.ds(s, 16)] = x1.astype(jnp.float32) - x
      wy_v[pl.ds(s, 16)] = y1.astype(jnp.float32) - y
      degen = (zf >= D - 1) | (zf < 0)
      wz_v[pl.ds(s, 16)] = jnp.where(
          degen, 1.0, (zf + 1).astype(jnp.float32) - z)
      cb_v[pl.ds(s, 16)] = (z0 & 1) << 1
      rz = z0 >> 1
      ty0 = y0 << 7
      ty1 = y1 << 7
      r = g // 8
      cix = (g % 8) * 16
      idx_v[0, r, pl.ds(cix, 16)] = ((bhw + ty0 + x0) << 6) + rz
      idx_v[1, r, pl.ds(cix, 16)] = ((bhw + ty1 + x0) << 6) + rz
      idx_v[2, r, pl.ds(cix, 16)] = ((bhw + ty0 + x1) << 6) + rz
      idx_v[3, r, pl.ds(cix, 16)] = ((bhw + ty1 + x1) << 6) + rz
      return 0

    lax.fori_loop(0, GROUPS, grp_idx, 0, unroll=4)

    for corner in range(4):
      for r in range(KC):
        pltpu.async_copy(table_hbm.at[idx_v.at[corner, r]],
                         rows_v.at[corner, pl.ds(r * 128, 128)], sem)
    for corner in range(4):
      for r in range(KC):
        pltpu.make_async_copy(table_hbm.at[idx_v.at[corner, r]],
                              rows_v.at[corner, pl.ds(r * 128, 128)],
                              sem).wait()

    def grp_sum(g, _):
      s = g * 16
      rows = s + lax.iota(jnp.int32, 16)
      c0 = cb_v[pl.ds(s, 16)]
      c1 = c0 + 1
      c2 = c0 + 2
      c3 = c0 + 3
      wx0 = wx_v[pl.ds(s, 16)]
      wy0 = wy_v[pl.ds(s, 16)]
      wz0 = wz_v[pl.ds(s, 16)]
      wx1 = 1.0 - wx0
      wy1 = 1.0 - wy0
      wz1 = 1.0 - wz0
      ws = (wy0 * wx0, wy1 * wx0, wy0 * wx1, wy1 * wx1)
      acc0 = jnp.zeros((16,), jnp.float32)
      acc1 = jnp.zeros((16,), jnp.float32)
      for corner in range(4):
        vz00 = plsc.load_gather(rows_v.at[corner], [rows, c0])
        vz01 = plsc.load_gather(rows_v.at[corner], [rows, c1])
        vz10 = plsc.load_gather(rows_v.at[corner], [rows, c2])
        vz11 = plsc.load_gather(rows_v.at[corner], [rows, c3])
        acc0 = acc0 + ws[corner] * (wz0 * vz00 + wz1 * vz10)
        acc1 = acc1 + ws[corner] * (wz0 * vz01 + wz1 * vz11)
      o0_v[pl.ds(s, 16)] = acc0
      o1_v[pl.ds(s, 16)] = acc1
      return 0

    lax.fori_loop(0, GROUPS, grp_sum, 0, unroll=4)

    pltpu.sync_copy(o0_v, out_hbm.at[pl.ds((b * C + 0) * HWD + o0, K)])
    pltpu.sync_copy(o1_v, out_hbm.at[pl.ds((b * C + 1) * HWD + o0, K)])
    return 0

  lax.fori_loop(0, NCHUNK, chunk_body, 0)


@jax.jit
def _warp(table, flow_flat):
  mesh = plsc.VectorSubcoreMesh(core_axis_name="c", subcore_axis_name="s")
  fn = pl.kernel(
      _warp_body,
      out_type=jax.ShapeDtypeStruct((B * C * HWD,), jnp.float32),
      mesh=mesh,
      compiler_params=pltpu.CompilerParams(
          needs_layout_passes=False, use_tc_tiling_on_sc=False),
      scratch_types=[
          pltpu.VMEM((K,), jnp.float32),
          pltpu.VMEM((K,), jnp.float32),
          pltpu.VMEM((K,), jnp.float32),
          pltpu.VMEM((K,), jnp.float32),
          pltpu.VMEM((K,), jnp.float32),
          pltpu.VMEM((K,), jnp.float32),
          pltpu.VMEM((K,), jnp.int32),
          pltpu.VMEM((4, KC, 128), jnp.int32),
          pltpu.VMEM((4, K, RW), jnp.float32),
          pltpu.VMEM((K,), jnp.float32),
          pltpu.VMEM((K,), jnp.float32),
          pltpu.SemaphoreType.DMA,
      ],
  )
  return fn(table, flow_flat)


def kernel(image, flow):
  vol = jnp.transpose(image, (0, 2, 3, 4, 1))
  volp = jnp.pad(vol, ((0, 0), (0, 0), (0, 0), (0, 2), (0, 0)))
  p4 = volp.reshape(B, H, W, (D + 2) // 2, 2 * C)
  tbl = jnp.concatenate([p4[:, :, :, :RZ, :], p4[:, :, :, 1:RZ + 1, :]],
                        axis=-1).reshape(NROW, RW)
  out = _warp(tbl, flow.reshape(-1))
  return out.reshape(B, C, H, W, D)
